# baseline (speedup 1.0000x reference)
import dataclasses
import functools

import jax
import jax.numpy as jnp
from jax import lax
from jax.experimental import pallas as pl
from jax.experimental.pallas import tpu as pltpu
from jax.experimental.pallas import tpu_sc as plsc

N_NODES = 10000
N_EDGES = 320000
C = 128
NEG_SLOPE = 0.2

N_PAD = 10240
TILES = 16
EPT = N_EDGES // TILES
LANES = 16
UNROLL = 5
SLICE_N = N_PAD // TILES


def _scores_tc(x, a2):
    def body(a_ref, x_ref, o_ref):
        res = jax.lax.dot_general(
            a_ref[...], x_ref[...],
            dimension_numbers=(((1,), (1,)), ((), ())),
            preferred_element_type=jnp.float32,
            precision=jax.lax.Precision.HIGHEST,
        )
        o_ref[pl.ds(0, N_NODES)] = res[0]
        o_ref[pl.ds(N_NODES, N_NODES)] = res[1]

    return pl.pallas_call(
        body,
        out_shape=jax.ShapeDtypeStruct((2 * N_NODES,), jnp.float32),
    )(a2, x)


def _edge_softmax_sc(scores2, edge_index):
    mesh = plsc.VectorSubcoreMesh(
        core_axis_name="c", subcore_axis_name="s", num_cores=1)
    cp = pltpu.CompilerParams()
    if "needs_layout_passes" in pltpu.CompilerParams.__dataclass_fields__:
        cp = dataclasses.replace(cp, needs_layout_passes=False)

    @functools.partial(
        pl.kernel,
        compiler_params=cp,
        out_type=jax.ShapeDtypeStruct((TILES, EPT), jnp.float32),
        mesh=mesh,
        scratch_types=[
            pltpu.VMEM((N_NODES,), jnp.float32),
            pltpu.VMEM((N_NODES,), jnp.float32),
            pltpu.VMEM((EPT,), jnp.int32),
            pltpu.VMEM((EPT,), jnp.int32),
            pltpu.VMEM((EPT,), jnp.float32),
            pltpu.VMEM((N_PAD,), jnp.float32),
            pltpu.VMEM((N_PAD,), jnp.float32),
            pltpu.VMEM((TILES, SLICE_N), jnp.float32),
            pltpu.VMEM_SHARED((TILES, N_PAD), jnp.float32),
            pltpu.VMEM_SHARED((N_PAD,), jnp.float32),
            pltpu.SemaphoreType.DMA,
        ],
    )
    def body(s_hbm, ei_hbm, out_hbm,
             s1_v, s2_v, row_v, col_v, ex_v, denom_p, recip_v, slab_v,
             part_sh, recip_sh, sem):
        wid = lax.axis_index("s")
        base = wid * EPT
        s1_sl = s_hbm.at[pl.ds(0, N_NODES)]
        s2_sl = s_hbm.at[pl.ds(N_NODES, N_NODES)]
        row_sl = ei_hbm.at[pl.ds(base, EPT)]
        col_sl = ei_hbm.at[pl.ds(N_EDGES + base, EPT)]
        pltpu.async_copy(s1_sl, s1_v, sem)
        pltpu.async_copy(s2_sl, s2_v, sem)
        pltpu.async_copy(row_sl, row_v, sem)
        pltpu.async_copy(col_sl, col_v, sem)

        zeros = jnp.zeros((LANES,), jnp.float32)

        @plsc.parallel_loop(0, N_PAD, step=LANES, unroll=8)
        def _(i):
            denom_p[pl.ds(i, LANES)] = zeros

        pltpu.make_async_copy(s1_sl, s1_v, sem).wait()
        pltpu.make_async_copy(s2_sl, s2_v, sem).wait()
        pltpu.make_async_copy(row_sl, row_v, sem).wait()
        pltpu.make_async_copy(col_sl, col_v, sem).wait()

        @plsc.parallel_loop(0, EPT, step=LANES, unroll=UNROLL)
        def _(j):
            sl = pl.ds(j, LANES)
            ir = row_v[sl]
            ic = col_v[sl]
            a = plsc.load_gather(s1_v, [ir]) + plsc.load_gather(s2_v, [ic])
            ex = jnp.exp(jnp.maximum(a, a * NEG_SLOPE))
            ex_v[sl] = ex
            plsc.addupdate_scatter(denom_p, [ir], ex)

        pltpu.sync_copy(denom_p, part_sh.at[wid])
        plsc.subcore_barrier()
        for t in range(TILES):
            pltpu.async_copy(
                part_sh.at[t, pl.ds(wid * SLICE_N, SLICE_N)], slab_v.at[t], sem)
        for t in range(TILES):
            pltpu.make_async_copy(
                part_sh.at[t, pl.ds(wid * SLICE_N, SLICE_N)], slab_v.at[t],
                sem).wait()

        @plsc.parallel_loop(0, SLICE_N, step=LANES, unroll=4)
        def _(v):
            sl = pl.ds(v, LANES)
            acc = slab_v[0, sl]
            for t in range(1, TILES):
                acc = acc + slab_v[t, sl]
            recip_v[sl] = 1.0 / acc

        pltpu.sync_copy(recip_v.at[pl.ds(0, SLICE_N)],
                        recip_sh.at[pl.ds(wid * SLICE_N, SLICE_N)])
        plsc.subcore_barrier()
        pltpu.sync_copy(recip_sh, recip_v)

        @plsc.parallel_loop(0, EPT, step=LANES, unroll=UNROLL)
        def _(j):
            sl = pl.ds(j, LANES)
            r = plsc.load_gather(recip_v, [row_v[sl]])
            ex_v[sl] = ex_v[sl] * r

        pltpu.sync_copy(ex_v, out_hbm.at[wid])

    return body(scores2, edge_index)


def kernel(x, edge_index, att):
    a2 = att[:, 0].reshape(2, C)
    s_flat = _scores_tc(x, a2)
    out = _edge_softmax_sc(s_flat, edge_index.reshape(2 * N_EDGES))
    return out.reshape(1, N_EDGES)
